# baseline (speedup 1.0000x reference)
import functools

import jax
import jax.numpy as jnp
from jax import lax
from jax.experimental import pallas as pl
from jax.experimental.pallas import tpu as pltpu
from jax.experimental.pallas import tpu_sc as plsc

_NC = 2
_NS = 16
_NW = _NC * _NS
_CHUNK = 128
_LANES = 16


@functools.partial(jax.jit, static_argnames=("batch", "dim"))
def _gather_add(h2, r2, table, *, batch, dim):
    bpw = batch // _NW
    n_chunks = bpw // _CHUNK
    vecs_per_row = dim // _LANES

    mesh = plsc.VectorSubcoreMesh(core_axis_name="c", subcore_axis_name="s")

    @functools.partial(
        pl.kernel,
        mesh=mesh,
        out_type=jax.ShapeDtypeStruct((batch, dim), jnp.float32),
        scratch_types=[
            pltpu.VMEM((n_chunks, _CHUNK), jnp.int32),
            pltpu.VMEM((n_chunks, _CHUNK), jnp.int32),
            pltpu.VMEM((bpw // 2, dim), jnp.float32),
            pltpu.VMEM((bpw // 2, dim), jnp.float32),
            pltpu.SemaphoreType.DMA,
            pltpu.SemaphoreType.DMA,
        ],
    )
    def k(h_hbm, r_hbm, table_hbm, out_hbm, hidx_v, ridx_v, rows_h, rows_r,
          semh, semr):
        wid = lax.axis_index("s") * _NC + lax.axis_index("c")
        pltpu.sync_copy(h_hbm.at[pl.ds(wid * n_chunks, n_chunks)], hidx_v)
        pltpu.sync_copy(r_hbm.at[pl.ds(wid * n_chunks, n_chunks)], ridx_v)

        groups_per_chunk = _CHUNK // _LANES
        hp = bpw // 2

        for half in range(2):
            gbase = half * (hp // _LANES)

            def fire(g, _):
                gg = gbase + g
                c = gg // groups_per_chunk
                o = (gg % groups_per_chunk) * _LANES
                hvec = hidx_v[c, pl.ds(o, _LANES)]
                rvec = ridx_v[c, pl.ds(o, _LANES)]
                base = g * _LANES
                for j in range(_LANES):
                    pltpu.async_copy(table_hbm.at[pl.ds(hvec[j], 1)],
                                     rows_h.at[pl.ds(base + j, 1)], semh)
                    pltpu.async_copy(table_hbm.at[pl.ds(rvec[j], 1)],
                                     rows_r.at[pl.ds(base + j, 1)], semr)
                return 0

            lax.fori_loop(0, hp // _LANES, fire, 0)
            pltpu.make_async_copy(table_hbm.at[pl.ds(0, hp)], rows_h,
                                  semh).wait()
            pltpu.make_async_copy(table_hbm.at[pl.ds(0, hp)], rows_r,
                                  semr).wait()

            def body(i, _):
                for v in range(vecs_per_row):
                    sl = pl.ds(v * _LANES, _LANES)
                    rows_h[i, sl] = rows_h[i, sl] + rows_r[i, sl]
                return 0

            lax.fori_loop(0, hp, body, 0)
            pltpu.sync_copy(rows_h,
                            out_hbm.at[pl.ds(wid * bpw + half * hp, hp)])

    return k(h2, r2, table)


def kernel(h_idx, r_idx, t_idx, table):
    del t_idx
    batch = h_idx.shape[0]
    dim = table.shape[1]
    h2 = h_idx.astype(jnp.int32).reshape(-1, _CHUNK)
    r2 = r_idx.astype(jnp.int32).reshape(-1, _CHUNK)
    return _gather_add(h2, r2, table, batch=batch, dim=dim)
